# baseline (speedup 1.0000x reference)
import jax
import jax.numpy as jnp
from jax.experimental import pallas as pl
from jax.experimental.pallas import tpu as pltpu

_N = 8192
_BLOCK_ROWS = 64
_GRID = _N // _BLOCK_ROWS
_NUMEL = float(_N) * float(_N)


def _body(p_ref, t_ref, o_ref, acc_ref):
    i = pl.program_id(0)

    @pl.when(i == 0)
    def _init():
        acc_ref[...] = jnp.zeros_like(acc_ref)

    d = jnp.clip(p_ref[...], 0.0, 1.0) - t_ref[...]
    acc_ref[...] += jnp.sum(d * d, axis=0, keepdims=True)

    @pl.when(i == _GRID - 1)
    def _fin():
        o_ref[0, 0] = jnp.sum(acc_ref[...]) * (1.0 / _NUMEL)


def kernel(pred, target):
    out = pl.pallas_call(
        _body,
        grid=(_GRID,),
        in_specs=[
            pl.BlockSpec((_BLOCK_ROWS, _N), lambda i: (i, 0)),
            pl.BlockSpec((_BLOCK_ROWS, _N), lambda i: (i, 0)),
        ],
        out_specs=pl.BlockSpec(memory_space=pltpu.SMEM),
        out_shape=jax.ShapeDtypeStruct((1, 1), jnp.float32),
        scratch_shapes=[pltpu.VMEM((1, _N), jnp.float32)],
        compiler_params=pltpu.CompilerParams(
            dimension_semantics=("arbitrary",),
        ),
    )(pred, target)
    return out[0, 0]


# speedup vs baseline: 1.1556x; 1.1556x over previous
import jax
import jax.numpy as jnp
from jax.experimental import pallas as pl
from jax.experimental.pallas import tpu as pltpu

_N = 8192
_BLOCK_ROWS = 128
_GRID = _N // _BLOCK_ROWS
_NUMEL = float(_N) * float(_N)


def _body(p_ref, t_ref, o_ref, acc_ref):
    i = pl.program_id(0)

    @pl.when(i == 0)
    def _init():
        acc_ref[...] = jnp.zeros_like(acc_ref)

    d = jnp.clip(p_ref[...], 0.0, 1.0) - t_ref[...]
    acc_ref[...] += jnp.sum(d * d, axis=0, keepdims=True)

    @pl.when(i == _GRID - 1)
    def _fin():
        o_ref[0, 0] = jnp.sum(acc_ref[...]) * (1.0 / _NUMEL)


def kernel(pred, target):
    out = pl.pallas_call(
        _body,
        grid=(_GRID,),
        in_specs=[
            pl.BlockSpec((_BLOCK_ROWS, _N), lambda i: (i, 0)),
            pl.BlockSpec((_BLOCK_ROWS, _N), lambda i: (i, 0)),
        ],
        out_specs=pl.BlockSpec(memory_space=pltpu.SMEM),
        out_shape=jax.ShapeDtypeStruct((1, 1), jnp.float32),
        scratch_shapes=[pltpu.VMEM((1, _N), jnp.float32)],
        compiler_params=pltpu.CompilerParams(
            dimension_semantics=("arbitrary",),
        ),
    )(pred, target)
    return out[0, 0]
